# Initial kernel scaffold; baseline (speedup 1.0000x reference)
import jax
import jax.numpy as jnp
from jax.experimental import pallas as pl


def kernel(x, edge_index, gat_W, gat_asrc, gat_adst, gW, gb, cW1, cb1, cW2, cb2, cW3, cb3, fW1, fb1, fW2, fb2, fW3, fb3):
    raise NotImplementedError("write your pallas kernel here")


# speedup vs baseline: 13.8604x; 13.8604x over previous
import functools

import jax
import jax.numpy as jnp
from jax import lax
from jax.experimental import pallas as pl
from jax.experimental.pallas import tpu as pltpu
from jax.experimental.pallas import tpu_sc as plsc

_N = 10000
_E = 320000
_D = 128
_NPAD = 10240
_NW = 32
_EPW = _E // _NW
_CH = 80
_NCH = _EPW // _CH
_RPS = _NPAD // 16


def _elu(x):
    return jnp.where(x > 0.0, x, jnp.exp(jnp.minimum(x, 0.0)) - 1.0)


def _combine(numa, numb, den):
    n = jnp.concatenate([numa[0] + numa[1], numb[0] + numb[1]], axis=1)
    d = den[0] + den[1] + 1e-16
    return _elu(n / d[:, None])


def _emit_pre(h, w_ref, asrc_ref, adst_ref, hwa_ref, hwb_ref, psd_ref):
    hw = jnp.dot(h, w_ref[...], preferred_element_type=jnp.float32)
    hwa_ref[...] = hw[:, :_D // 2]
    hwb_ref[...] = hw[:, _D // 2:]
    ps = jnp.sum(hw * asrc_ref[...], axis=1)
    pd = jnp.sum(hw * adst_ref[...], axis=1)
    psd_ref[...] = jnp.stack([ps, pd], axis=0)


def _tc_first_body(x_ref, w_ref, asrc_ref, adst_ref, hwa_ref, hwb_ref,
                   psd_ref):
    _emit_pre(x_ref[...], w_ref, asrc_ref, adst_ref, hwa_ref, hwb_ref, psd_ref)


def _tc_mid_body(numa_ref, numb_ref, den_ref, w_ref, asrc_ref, adst_ref,
                 hwa_ref, hwb_ref, psd_ref):
    h = _combine(numa_ref[:, :_N], numb_ref[:, :_N], den_ref[:, :_N])
    _emit_pre(h, w_ref, asrc_ref, adst_ref, hwa_ref, hwb_ref, psd_ref)


def _tc_head_body(numa_ref, numb_ref, den_ref, gw_ref, gb_ref, cw1a_ref,
                  cw1b_ref, cb1_ref, cw2_ref, cb2_ref, cw3_ref, cb3_ref,
                  fw1_ref, fb1_ref, fw2_ref, fb2_ref, fw3_ref, fb3_ref,
                  q_ref):
    h = _combine(numa_ref[:, :_N], numb_ref[:, :_N], den_ref[:, :_N])
    g = jnp.sum(h, axis=0, keepdims=True)
    v = g
    for i in range(3):
        v = _elu(jnp.dot(v, gw_ref[i], preferred_element_type=jnp.float32)
                 + gb_ref[i][None, :])
    a = h + v
    b = h * v
    z = _elu(jnp.dot(a, cw1a_ref[...], preferred_element_type=jnp.float32)
             + jnp.dot(b, cw1b_ref[...], preferred_element_type=jnp.float32)
             + cb1_ref[...])
    z = _elu(jnp.dot(z, cw2_ref[...], preferred_element_type=jnp.float32)
             + cb2_ref[...])
    z = jnp.dot(z, cw3_ref[...], preferred_element_type=jnp.float32) + cb3_ref[...]
    q = _elu(jnp.dot(z, fw1_ref[...], preferred_element_type=jnp.float32)
             + fb1_ref[...])
    q = _elu(jnp.dot(q, fw2_ref[...], preferred_element_type=jnp.float32)
             + fb2_ref[...])
    q_ref[...] = jnp.dot(q, fw3_ref[...], preferred_element_type=jnp.float32) + fb3_ref[...]


_PRE_OUT = [jax.ShapeDtypeStruct((_N, _D // 2), jnp.float32),
            jax.ShapeDtypeStruct((_N, _D // 2), jnp.float32),
            jax.ShapeDtypeStruct((2, _N), jnp.float32)]


def _tc_first(x, w, asrc, adst):
    return pl.pallas_call(_tc_first_body, out_shape=_PRE_OUT)(
        x, w, asrc, adst)


def _tc_mid(numa, numb, den, w, asrc, adst):
    return pl.pallas_call(_tc_mid_body, out_shape=_PRE_OUT)(
        numa, numb, den, w, asrc, adst)


def _tc_head(numa, numb, den, gw, gb, cw1a, cw1b, cb1, cw2, cb2, cw3, cb3,
             fw1, fb1, fw2, fb2, fw3, fb3):
    return pl.pallas_call(
        _tc_head_body,
        out_shape=jax.ShapeDtypeStruct((_N, 1), jnp.float32),
    )(numa, numb, den, gw, gb, cw1a, cw1b, cb1, cw2, cb2, cw3, cb3,
      fw1, fb1, fw2, fb2, fw3, fb3)


_DH = _D // 2


def _sc_edge_body(src_hbm, dst_hbm, ps_hbm, pd_hbm, hwa_hbm,
                  hwb_hbm, numa_hbm, numb_hbm, den_hbm,
                  src_v, dst_v, sidx, didx, ps_v, pd_v, ee_v, rowbuf, zrow,
                  zden, num_sh, den_sh, gsem):
    c = lax.axis_index("c")
    s = lax.axis_index("s")
    wid = c * 16 + s
    base = wid * _EPW

    def _zrow_body(i, carry):
        for u in range(_DH // 16):
            zrow[i, pl.ds(u * 16, 16)] = jnp.zeros((16,), jnp.float32)
        return carry
    lax.fori_loop(0, 128, _zrow_body, 0)

    def _zden_body(i, carry):
        zden[pl.ds(i * 16, 16)] = jnp.zeros((16,), jnp.float32)
        return carry
    lax.fori_loop(0, 40, _zden_body, 0)

    pltpu.sync_copy(src_hbm.at[pl.ds(base, _EPW)], src_v)
    pltpu.sync_copy(dst_hbm.at[pl.ds(base, _EPW)], dst_v)
    pltpu.sync_copy(ps_hbm, ps_v)
    pltpu.sync_copy(pd_hbm, pd_v)

    def _zero_accum():
        for jj in range(5):
            pltpu.sync_copy(zrow, num_sh.at[pl.ds(s * _RPS + jj * 128, 128)])

    def _pass(hw_hbm, num_hbm, first):
        def _chunk(j, carry):
            off = j * _CH
            for grp in range(_CH // 16):
                o = off + grp * 16
                s16 = src_v[pl.ds(o, 16)]
                d16 = dst_v[pl.ds(o, 16)]
                sidx[pl.ds(grp * 16, 16)] = s16
                didx[pl.ds(grp * 16, 16)] = d16
            gcp = pltpu.async_copy(hw_hbm.at[sidx], rowbuf, gsem)
            if first:
                for grp in range(_CH // 16):
                    o = off + grp * 16
                    s16 = src_v[pl.ds(o, 16)]
                    d16 = dst_v[pl.ds(o, 16)]
                    a = plsc.load_gather(ps_v, [s16])
                    b = plsc.load_gather(pd_v, [d16])
                    e = a + b
                    e = jnp.where(e >= 0.0, e, e * 0.2)
                    ee_v[pl.ds(o, 16)] = jnp.exp(e)
            gcp.wait()
            for r in range(_CH):
                av = plsc.load_gather(ee_v, [jnp.full((16,), r, jnp.int32) + off])
                for u in range(_DH // 16):
                    rowbuf[r, pl.ds(u * 16, 16)] = (
                        rowbuf[r, pl.ds(u * 16, 16)] * av)
            pltpu.sync_copy(rowbuf, num_sh.at[didx], add=True)
            if first:
                pltpu.sync_copy(ee_v.at[pl.ds(off, _CH)],
                                den_sh.at[didx], add=True)
            return carry
        lax.fori_loop(0, _NCH, _chunk, 0)
        plsc.subcore_barrier()
        pltpu.sync_copy(num_sh.at[pl.ds(s * _RPS, _RPS)],
                        num_hbm.at[c, pl.ds(s * _RPS, _RPS)])
        if first:
            pltpu.sync_copy(den_sh.at[pl.ds(s * 640, 640)],
                            den_hbm.at[c, pl.ds(s * 640, 640)])
        plsc.subcore_barrier()

    _zero_accum()
    pltpu.sync_copy(zden, den_sh.at[pl.ds(s * 640, 640)])
    plsc.subcore_barrier()
    _pass(hwa_hbm, numa_hbm, True)
    _zero_accum()
    plsc.subcore_barrier()
    _pass(hwb_hbm, numb_hbm, False)


@functools.partial(
    pl.kernel,
    mesh=plsc.VectorSubcoreMesh(core_axis_name="c", subcore_axis_name="s"),
    compiler_params=pltpu.CompilerParams(needs_layout_passes=False,
                                         use_tc_tiling_on_sc=False),
    out_type=[jax.ShapeDtypeStruct((2, _NPAD, _DH), jnp.float32),
              jax.ShapeDtypeStruct((2, _NPAD, _DH), jnp.float32),
              jax.ShapeDtypeStruct((2, _NPAD), jnp.float32)],
    scratch_types=[
        pltpu.VMEM((_EPW,), jnp.int32),
        pltpu.VMEM((_EPW,), jnp.int32),
        pltpu.VMEM((_CH,), jnp.int32),
        pltpu.VMEM((_CH,), jnp.int32),
        pltpu.VMEM((_N,), jnp.float32),
        pltpu.VMEM((_N,), jnp.float32),
        pltpu.VMEM((_EPW,), jnp.float32),
        pltpu.VMEM((_CH, _DH), jnp.float32),
        pltpu.VMEM((128, _DH), jnp.float32),
        pltpu.VMEM((640,), jnp.float32),
        pltpu.VMEM_SHARED((_NPAD, _DH), jnp.float32),
        pltpu.VMEM_SHARED((_NPAD,), jnp.float32),
        pltpu.SemaphoreType.DMA,
    ],
)
def _sc_edge(src_hbm, dst_hbm, ps_hbm, pd_hbm, hwa_hbm, hwb_hbm,
             numa_hbm, numb_hbm, den_hbm, *scratch):
    _sc_edge_body(src_hbm, dst_hbm, ps_hbm, pd_hbm, hwa_hbm,
                  hwb_hbm, numa_hbm, numb_hbm, den_hbm, *scratch)


def _impl(x, edge_index, gat_W, gat_asrc, gat_adst, gW, gb, cW1, cb1,
          cW2, cb2, cW3, cb3, fW1, fb1, fW2, fb2, fW3, fb3):
    src = edge_index[0]
    dst = edge_index[1]

    hwa, hwb, psd = _tc_first(x, gat_W[0], gat_asrc[0][None, :],
                              gat_adst[0][None, :])
    numa, numb, den = _sc_edge(src, dst, psd[0], psd[1], hwa, hwb)
    for l in range(1, 3):
        hwa, hwb, psd = _tc_mid(numa, numb, den, gat_W[l],
                                gat_asrc[l][None, :], gat_adst[l][None, :])
        numa, numb, den = _sc_edge(src, dst, psd[0], psd[1], hwa, hwb)

    q = _tc_head(numa, numb, den, gW, gb, cW1[:_D], cW1[_D:], cb1[None, :],
                 cW2, cb2[None, :], cW3, cb3[None, :],
                 fW1, fb1[None, :], fW2, fb2[None, :], fW3, fb3[None, :])
    return q


def kernel(x, edge_index, gat_W, gat_asrc, gat_adst, gW, gb, cW1, cb1,
           cW2, cb2, cW3, cb3, fW1, fb1, fW2, fb2, fW3, fb3):
    return _impl(x, edge_index, gat_W, gat_asrc, gat_adst, gW, gb, cW1, cb1,
                 cW2, cb2, cW3, cb3, fW1, fb1, fW2, fb2, fW3, fb3)
